# baseline (speedup 1.0000x reference)
import math

import jax
import jax.numpy as jnp
from jax import lax
from jax.experimental import pallas as pl
from jax.experimental.pallas import tpu as pltpu
from jax.experimental.pallas import tpu_sc as plsc

_SIZE = 100000
_PAD = 0
_SMOOTHING = 0.1
_CONF = 1.0 - _SMOOTHING
_EPS = _SMOOTHING / (_SIZE - 2)
_C1 = _SMOOTHING * math.log(_EPS) + _CONF * math.log(_CONF)

_B = 1024

_NC, _NS, _L = 2, 16, 16
_NW = _NC * _NS

_B_SC = 512
_B_TC = _B - _B_SC
_RPT = _B_SC // _NW

_CSPAN = 98304
_CW = 6144
_NCH = _CSPAN // _CW
_UNROLL = 4
_KSTEP = _CW // _L // _UNROLL

_TAILW = 2048
_TAIL0 = _CSPAN // _TAILW

_R = 32
_NRB = _B_TC // _R
_GPS = _B_SC // _NRB


def _sc_body(x_hbm, out_hbm, buf0, buf1, sums_v, sem0, sem1):
    wid = lax.axis_index("s") * _NC + lax.axis_index("c")
    tile_row0 = _B_TC + wid * _RPT

    def chunk_copy(grp_row0, c, buf, sem):
        return pltpu.make_async_copy(
            x_hbm.at[pl.ds(grp_row0, 8), pl.ds(c * _CW, _CW)], buf, sem)

    def rows_add(buf, accs):
        out = []
        for r in range(8):
            def inner(k, acc, r=r):
                o = pl.multiple_of(k * (_UNROLL * _L), _L)
                for u in range(_UNROLL):
                    acc = acc + buf[r, pl.ds(o + u * _L, _L)]
                return acc

            out.append(lax.fori_loop(0, _KSTEP, inner, accs[r]))
        return tuple(out)

    for g in range(2):
        grp_row0 = tile_row0 + g * 8
        chunk_copy(grp_row0, 0, buf0, sem0).start()

        def pair_body(k, a, grp_row0=grp_row0):
            q0 = k * 2
            chunk_copy(grp_row0, q0 + 1, buf1, sem1).start()
            chunk_copy(grp_row0, q0, buf0, sem0).wait()
            a = rows_add(buf0, a)

            @pl.when(q0 + 2 < _NCH)
            def _nxt():
                chunk_copy(grp_row0, q0 + 2, buf0, sem0).start()

            chunk_copy(grp_row0, q0 + 1, buf1, sem1).wait()
            return rows_add(buf1, a)

        grp_accs = lax.fori_loop(
            0, _NCH // 2, pair_body,
            tuple(jnp.zeros((_L,), jnp.float32) for _ in range(8)))
        for r in range(8):
            sums_v[g * 8 + r, pl.ds(0, _L)] = grp_accs[r]

    pltpu.sync_copy(sums_v, out_hbm.at[pl.ds(wid * _RPT, _RPT), :])


_sc_partials_cache = None


def _get_sc_partials():
    global _sc_partials_cache
    if _sc_partials_cache is None:
        _sc_partials_cache = pl.kernel(
            _sc_body,
            out_type=jax.ShapeDtypeStruct((_B_SC, _L), jnp.float32),
            mesh=plsc.VectorSubcoreMesh(
                core_axis_name="c", subcore_axis_name="s", num_cores=_NC,
                num_subcores=_NS),
            scratch_types=[
                pltpu.VMEM((8, _CW), jnp.float32),
                pltpu.VMEM((8, _CW), jnp.float32),
                pltpu.VMEM((_RPT, _L), jnp.float32),
                pltpu.SemaphoreType.DMA,
                pltpu.SemaphoreType.DMA,
            ],
        )
    return _sc_partials_cache


def _loss_body(tpref, t_ref, tsc_ref, sp_ref, xtail_ref, xcol0_ref, x_ref,
               *g_and_out):
    g_refs = g_and_out[:_GPS]
    o_ref = g_and_out[_GPS]
    i = pl.program_id(0)
    x = x_ref[...]
    t = t_ref[0]
    col = jax.lax.broadcasted_iota(jnp.int32, (_R, _SIZE), 1)
    s = jnp.sum(x, axis=1, keepdims=True) - x[:, 0:1]
    g = jnp.sum(jnp.where(col == t, x, 0.0), axis=1, keepdims=True)
    row_term = _C1 - _EPS * s + (_EPS - _CONF) * g
    partial = jnp.sum(jnp.where(t != _PAD, row_term, 0.0))

    lane = jax.lax.broadcasted_iota(jnp.int32, (1, 128), 1)
    for j in range(_GPS):
        tb = tpref[_B_TC + i * _GPS + j]
        colbase = (tb // 128) * 128
        blk = g_refs[j][pl.ds(j % 8, 1), :]
        gj = jnp.sum(jnp.where(lane + colbase == tb, blk, 0.0))
        partial += jnp.where(tb != _PAD, (_EPS - _CONF) * gj, 0.0)

    @pl.when(i == 0)
    def _init():
        o_ref[0, 0] = 0.0

    @pl.when(i < _NRB - 1)
    def _mid():
        o_ref[0, 0] += partial * (1.0 / _B)

    @pl.when(i == _NRB - 1)
    def _last():
        ts = tsc_ref[0]
        xt = xtail_ref[...]
        x0 = xcol0_ref[...][:, 0:1]
        s_sc = jnp.sum(sp_ref[...], axis=1, keepdims=True)
        colt = jax.lax.broadcasted_iota(
            jnp.int32, (_B_SC, _TAILW), 1) + _CSPAN
        s_t = jnp.sum(jnp.where(colt < _SIZE, xt, 0.0), axis=1,
                      keepdims=True)
        term = _C1 - _EPS * (s_sc + s_t - x0)
        sc_fix = jnp.sum(jnp.where(ts != _PAD, term, 0.0))
        o_ref[0, 0] += (partial + sc_fix) * (1.0 / _B)


def _gather_spec(j):
    def idx(i, tpref):
        b = _B_TC + i * _GPS + j
        return (b // 8, tpref[b] // 128)

    return pl.BlockSpec((8, 128), idx)


@jax.jit
def kernel(x, target):
    t32 = target.astype(jnp.int32)
    scpart = _get_sc_partials()(x)
    t3 = t32[:_B_TC].reshape(_NRB, _R, 1)
    tsc3 = t32[_B_TC:].reshape(1, _B_SC, 1)
    grid_spec = pltpu.PrefetchScalarGridSpec(
        num_scalar_prefetch=1,
        grid=(_NRB,),
        in_specs=[
            pl.BlockSpec((1, _R, 1), lambda i, tp: (i, 0, 0)),
            pl.BlockSpec((1, _B_SC, 1), lambda i, tp: (0, 0, 0)),
            pl.BlockSpec((_B_SC, _L), lambda i, tp: (0, 0)),
            pl.BlockSpec((_B_SC, _TAILW), lambda i, tp: (1, _TAIL0)),
            pl.BlockSpec((_B_SC, 128), lambda i, tp: (1, 0)),
            pl.BlockSpec((_R, _SIZE), lambda i, tp: (i, 0)),
        ] + [_gather_spec(j) for j in range(_GPS)],
        out_specs=pl.BlockSpec(memory_space=pltpu.SMEM),
    )
    out = pl.pallas_call(
        _loss_body,
        grid_spec=grid_spec,
        out_shape=jax.ShapeDtypeStruct((1, 1), jnp.float32),
    )(t32, t3, tsc3, scpart, x, x, x, *([x] * _GPS))
    return out[0, 0]


# speedup vs baseline: 1.1122x; 1.1122x over previous
import math

import jax
import jax.numpy as jnp
from jax import lax
from jax.experimental import pallas as pl
from jax.experimental.pallas import tpu as pltpu
from jax.experimental.pallas import tpu_sc as plsc

_SIZE = 100000
_PAD = 0
_SMOOTHING = 0.1
_CONF = 1.0 - _SMOOTHING
_EPS = _SMOOTHING / (_SIZE - 2)
_C1 = _SMOOTHING * math.log(_EPS) + _CONF * math.log(_CONF)

_B = 1024

_NC, _NS, _L = 2, 16, 16
_NW = _NC * _NS

_B_SC = 512
_B_TC = _B - _B_SC
_RPT = _B_SC // _NW

_CSPAN = 98304
_CW = 6144
_NCH = _CSPAN // _CW
_UNROLL = 4
_KSTEP = _CW // _L // _UNROLL

_TAILW = 2048
_TAIL0 = _CSPAN // _TAILW

_R = 32
_NRB = _B_TC // _R
_GPS = _B_SC // _NRB


def _dyn_gather(vec, idx):
    return lax.gather(
        vec, idx.reshape(_L, 1),
        lax.GatherDimensionNumbers(
            offset_dims=(), collapsed_slice_dims=(0,), start_index_map=(0,)),
        (1,),
        mode=lax.GatherScatterMode.PROMISE_IN_BOUNDS)


def _sc_body(t_hbm, x_hbm, out_hbm, t_v, buf0, buf1, sums_v, sem0, sem1):
    wid = lax.axis_index("s") * _NC + lax.axis_index("c")
    tile_row0 = _B_TC + wid * _RPT
    pltpu.sync_copy(t_hbm.at[pl.ds(wid * _RPT, _RPT)], t_v)
    tv = t_v[...]
    iot = lax.iota(jnp.int32, _L)
    npv = jnp.where(tv != _PAD, 1.0, 0.0)

    def chunk_copy(grp_row0, c, buf, sem):
        return pltpu.make_async_copy(
            x_hbm.at[pl.ds(grp_row0, 8), pl.ds(c * _CW, _CW)], buf, sem)

    def rows_add(buf, accs):
        out = []
        for r in range(8):
            def inner(k, acc, r=r):
                o = pl.multiple_of(k * (_UNROLL * _L), _L)
                for u in range(_UNROLL):
                    acc = acc + buf[r, pl.ds(o + u * _L, _L)]
                return acc

            out.append(lax.fori_loop(0, _KSTEP, inner, accs[r]))
        return tuple(out)

    for g in range(2):
        grp_row0 = tile_row0 + g * 8
        chunk_copy(grp_row0, 0, buf0, sem0).start()

        def pair_body(k, a, grp_row0=grp_row0):
            q0 = k * 2
            chunk_copy(grp_row0, q0 + 1, buf1, sem1).start()
            chunk_copy(grp_row0, q0, buf0, sem0).wait()
            a = rows_add(buf0, a)

            @pl.when(q0 + 2 < _NCH)
            def _nxt():
                chunk_copy(grp_row0, q0 + 2, buf0, sem0).start()

            chunk_copy(grp_row0, q0 + 1, buf1, sem1).wait()
            return rows_add(buf1, a)

        grp_accs = lax.fori_loop(
            0, _NCH // 2, pair_body,
            tuple(jnp.zeros((_L,), jnp.float32) for _ in range(8)))
        for r in range(8):
            mrow = _dyn_gather(npv, iot * 0 + (g * 8 + r))
            sums_v[g * 8 + r, pl.ds(0, _L)] = mrow * grp_accs[r]

    pltpu.sync_copy(sums_v, out_hbm.at[pl.ds(wid * _RPT, _RPT), :])


_sc_partials_cache = None


def _get_sc_partials():
    global _sc_partials_cache
    if _sc_partials_cache is None:
        _sc_partials_cache = pl.kernel(
            _sc_body,
            out_type=jax.ShapeDtypeStruct((_B_SC, _L), jnp.float32),
            mesh=plsc.VectorSubcoreMesh(
                core_axis_name="c", subcore_axis_name="s", num_cores=_NC,
                num_subcores=_NS),
            scratch_types=[
                pltpu.VMEM((_RPT,), jnp.int32),
                pltpu.VMEM((8, _CW), jnp.float32),
                pltpu.VMEM((8, _CW), jnp.float32),
                pltpu.VMEM((_RPT, _L), jnp.float32),
                pltpu.SemaphoreType.DMA,
                pltpu.SemaphoreType.DMA,
            ],
        )
    return _sc_partials_cache


def _combine_body(sp_ref, o_ref):
    o_ref[0, 0] = jnp.sum(sp_ref[...]) * (-_EPS / _B)


def _loss_body(tpref, t_ref, tsc_ref, xtail_ref, xcol0_ref, x_ref,
               *g_and_out):
    g_refs = g_and_out[:_GPS]
    o_ref = g_and_out[_GPS]
    i = pl.program_id(0)
    x = x_ref[...]
    t = t_ref[0]
    col = jax.lax.broadcasted_iota(jnp.int32, (_R, _SIZE), 1)
    s = jnp.sum(x, axis=1, keepdims=True) - x[:, 0:1]
    g = jnp.sum(jnp.where(col == t, x, 0.0), axis=1, keepdims=True)
    row_term = _C1 - _EPS * s + (_EPS - _CONF) * g
    partial = jnp.sum(jnp.where(t != _PAD, row_term, 0.0))

    lane = jax.lax.broadcasted_iota(jnp.int32, (1, 128), 1)
    for j in range(_GPS):
        tb = tpref[_B_TC + i * _GPS + j]
        colbase = (tb // 128) * 128
        blk = g_refs[j][pl.ds(j % 8, 1), :]
        gj = jnp.sum(jnp.where(lane + colbase == tb, blk, 0.0))
        partial += jnp.where(tb != _PAD, (_EPS - _CONF) * gj, 0.0)

    @pl.when(i == 0)
    def _init():
        o_ref[0, 0] = 0.0

    @pl.when(i < _NRB - 1)
    def _mid():
        o_ref[0, 0] += partial * (1.0 / _B)

    @pl.when(i == _NRB - 1)
    def _last():
        ts = tsc_ref[0]
        xt = xtail_ref[...]
        x0 = xcol0_ref[...][:, 0:1]
        colt = jax.lax.broadcasted_iota(
            jnp.int32, (_B_SC, _TAILW), 1) + _CSPAN
        s_t = jnp.sum(jnp.where(colt < _SIZE, xt, 0.0), axis=1,
                      keepdims=True)
        term = _C1 - _EPS * (s_t - x0)
        sc_fix = jnp.sum(jnp.where(ts != _PAD, term, 0.0))
        o_ref[0, 0] += (partial + sc_fix) * (1.0 / _B)


def _gather_spec(j):
    def idx(i, tpref):
        b = _B_TC + i * _GPS + j
        return (b // 8, tpref[b] // 128)

    return pl.BlockSpec((8, 128), idx)


@jax.jit
def kernel(x, target):
    t32 = target.astype(jnp.int32)
    scpart = _get_sc_partials()(t32[_B_TC:], x)
    sc_sum = pl.pallas_call(
        _combine_body,
        in_specs=[pl.BlockSpec((_B_SC, _L), lambda: (0, 0))],
        out_specs=pl.BlockSpec(memory_space=pltpu.SMEM),
        out_shape=jax.ShapeDtypeStruct((1, 1), jnp.float32),
    )(scpart)
    t3 = t32[:_B_TC].reshape(_NRB, _R, 1)
    tsc3 = t32[_B_TC:].reshape(1, _B_SC, 1)
    grid_spec = pltpu.PrefetchScalarGridSpec(
        num_scalar_prefetch=1,
        grid=(_NRB,),
        in_specs=[
            pl.BlockSpec((1, _R, 1), lambda i, tp: (i, 0, 0)),
            pl.BlockSpec((1, _B_SC, 1), lambda i, tp: (0, 0, 0)),
            pl.BlockSpec((_B_SC, _TAILW), lambda i, tp: (1, _TAIL0)),
            pl.BlockSpec((_B_SC, 128), lambda i, tp: (1, 0)),
            pl.BlockSpec((_R, _SIZE), lambda i, tp: (i, 0)),
        ] + [_gather_spec(j) for j in range(_GPS)],
        out_specs=pl.BlockSpec(memory_space=pltpu.SMEM),
    )
    out = pl.pallas_call(
        _loss_body,
        grid_spec=grid_spec,
        out_shape=jax.ShapeDtypeStruct((1, 1), jnp.float32),
    )(t32, t3, tsc3, x, x, x, *([x] * _GPS))
    return out[0, 0] + sc_sum[0, 0]


# speedup vs baseline: 1.1172x; 1.0045x over previous
import math

import jax
import jax.numpy as jnp
from jax import lax
from jax.experimental import pallas as pl
from jax.experimental.pallas import tpu as pltpu
from jax.experimental.pallas import tpu_sc as plsc

_SIZE = 100000
_PAD = 0
_SMOOTHING = 0.1
_CONF = 1.0 - _SMOOTHING
_EPS = _SMOOTHING / (_SIZE - 2)
_C1 = _SMOOTHING * math.log(_EPS) + _CONF * math.log(_CONF)

_B = 1024

_NC, _NS, _L = 2, 16, 16
_NW = _NC * _NS

_B_SC = 512
_B_TC = _B - _B_SC
_RPT = _B_SC // _NW

_CSPAN = 98304
_CW = 6144
_NCH = _CSPAN // _CW
_UNROLL = 4
_KSTEP = _CW // _L // _UNROLL

_TAILW = 2048
_TAIL0 = _CSPAN // _TAILW

_R = 32
_NRB = _B_TC // _R
_GPS = _B_SC // _NRB


def _dyn_gather(vec, idx):
    return lax.gather(
        vec, idx.reshape(_L, 1),
        lax.GatherDimensionNumbers(
            offset_dims=(), collapsed_slice_dims=(0,), start_index_map=(0,)),
        (1,),
        mode=lax.GatherScatterMode.PROMISE_IN_BOUNDS)


def _sc_body(t_hbm, x_hbm, out_hbm, t_v, buf0, buf1, sums_v, sem0, sem1):
    wid = lax.axis_index("s") * _NC + lax.axis_index("c")
    tile_row0 = _B_TC + wid * _RPT
    pltpu.sync_copy(t_hbm.at[pl.ds(wid * _RPT, _RPT)], t_v)
    tv = t_v[...]
    iot = lax.iota(jnp.int32, _L)
    npv = jnp.where(tv != _PAD, 1.0, 0.0)

    def chunk_copy(grp_row0, c, buf, sem):
        return pltpu.make_async_copy(
            x_hbm.at[pl.ds(grp_row0, 8), pl.ds(c * _CW, _CW)], buf, sem)

    def rows_add(buf, accs):
        out = []
        for r in range(8):
            def inner(k, acc, r=r):
                o = pl.multiple_of(k * (_UNROLL * _L), _L)
                for u in range(_UNROLL):
                    acc = acc + buf[r, pl.ds(o + u * _L, _L)]
                return acc

            out.append(lax.fori_loop(0, _KSTEP, inner, accs[r]))
        return tuple(out)

    for g in range(2):
        grp_row0 = tile_row0 + g * 8
        chunk_copy(grp_row0, 0, buf0, sem0).start()

        def pair_body(k, a, grp_row0=grp_row0):
            q0 = k * 2
            chunk_copy(grp_row0, q0 + 1, buf1, sem1).start()
            chunk_copy(grp_row0, q0, buf0, sem0).wait()
            a = rows_add(buf0, a)

            @pl.when(q0 + 2 < _NCH)
            def _nxt():
                chunk_copy(grp_row0, q0 + 2, buf0, sem0).start()

            chunk_copy(grp_row0, q0 + 1, buf1, sem1).wait()
            return rows_add(buf1, a)

        grp_accs = lax.fori_loop(
            0, _NCH // 2, pair_body,
            tuple(jnp.zeros((_L,), jnp.float32) for _ in range(8)))
        for r in range(8):
            mrow = _dyn_gather(npv, iot * 0 + (g * 8 + r))
            sums_v[g * 8 + r, pl.ds(0, _L)] = mrow * grp_accs[r]

    pltpu.sync_copy(sums_v, out_hbm.at[pl.ds(wid * _RPT, _RPT), :])


_sc_partials_cache = None


def _get_sc_partials():
    global _sc_partials_cache
    if _sc_partials_cache is None:
        _sc_partials_cache = pl.kernel(
            _sc_body,
            out_type=jax.ShapeDtypeStruct((_B_SC, _L), jnp.float32),
            mesh=plsc.VectorSubcoreMesh(
                core_axis_name="c", subcore_axis_name="s", num_cores=_NC,
                num_subcores=_NS),
            scratch_types=[
                pltpu.VMEM((_RPT,), jnp.int32),
                pltpu.VMEM((8, _CW), jnp.float32),
                pltpu.VMEM((8, _CW), jnp.float32),
                pltpu.VMEM((_RPT, _L), jnp.float32),
                pltpu.SemaphoreType.DMA,
                pltpu.SemaphoreType.DMA,
            ],
        )
    return _sc_partials_cache


def _combine_body(main_ref, sp_ref, o_ref):
    o_ref[0, 0] = main_ref[0, 0] + jnp.sum(sp_ref[...]) * (-_EPS / _B)


def _loss_body(tpref, t_ref, tsc_ref, xtail_ref, xcol0_ref, x_ref,
               *g_and_out):
    g_refs = g_and_out[:_GPS]
    o_ref = g_and_out[_GPS]
    i = pl.program_id(0)
    x = x_ref[...]
    t = t_ref[0]
    col = jax.lax.broadcasted_iota(jnp.int32, (_R, _SIZE), 1)
    s = jnp.sum(x, axis=1, keepdims=True) - x[:, 0:1]
    g = jnp.sum(jnp.where(col == t, x, 0.0), axis=1, keepdims=True)
    row_term = _C1 - _EPS * s + (_EPS - _CONF) * g
    partial = jnp.sum(jnp.where(t != _PAD, row_term, 0.0))

    lane = jax.lax.broadcasted_iota(jnp.int32, (1, 128), 1)
    for j in range(_GPS):
        tb = tpref[_B_TC + i * _GPS + j]
        colbase = (tb // 128) * 128
        blk = g_refs[j][pl.ds(j % 8, 1), :]
        gj = jnp.sum(jnp.where(lane + colbase == tb, blk, 0.0))
        partial += jnp.where(tb != _PAD, (_EPS - _CONF) * gj, 0.0)

    @pl.when(i == 0)
    def _init():
        o_ref[0, 0] = 0.0

    @pl.when(i < _NRB - 1)
    def _mid():
        o_ref[0, 0] += partial * (1.0 / _B)

    @pl.when(i == _NRB - 1)
    def _last():
        ts = tsc_ref[0]
        xt = xtail_ref[...]
        x0 = xcol0_ref[...][:, 0:1]
        colt = jax.lax.broadcasted_iota(
            jnp.int32, (_B_SC, _TAILW), 1) + _CSPAN
        s_t = jnp.sum(jnp.where(colt < _SIZE, xt, 0.0), axis=1,
                      keepdims=True)
        term = _C1 - _EPS * (s_t - x0)
        sc_fix = jnp.sum(jnp.where(ts != _PAD, term, 0.0))
        o_ref[0, 0] += (partial + sc_fix) * (1.0 / _B)


def _gather_spec(j):
    def idx(i, tpref):
        b = _B_TC + i * _GPS + j
        return (b // 8, tpref[b] // 128)

    return pl.BlockSpec((8, 128), idx)


@jax.jit
def kernel(x, target):
    t32 = target.astype(jnp.int32)
    scpart = _get_sc_partials()(t32[_B_TC:], x)
    t3 = t32[:_B_TC].reshape(_NRB, _R, 1)
    tsc3 = t32[_B_TC:].reshape(1, _B_SC, 1)
    grid_spec = pltpu.PrefetchScalarGridSpec(
        num_scalar_prefetch=1,
        grid=(_NRB,),
        in_specs=[
            pl.BlockSpec((1, _R, 1), lambda i, tp: (i, 0, 0)),
            pl.BlockSpec((1, _B_SC, 1), lambda i, tp: (0, 0, 0)),
            pl.BlockSpec((_B_SC, _TAILW), lambda i, tp: (1, _TAIL0)),
            pl.BlockSpec((_B_SC, 128), lambda i, tp: (1, 0)),
            pl.BlockSpec((_R, _SIZE), lambda i, tp: (i, 0)),
        ] + [_gather_spec(j) for j in range(_GPS)],
        out_specs=pl.BlockSpec(memory_space=pltpu.SMEM),
    )
    out = pl.pallas_call(
        _loss_body,
        grid_spec=grid_spec,
        out_shape=jax.ShapeDtypeStruct((1, 1), jnp.float32),
    )(t32, t3, tsc3, x, x, x, *([x] * _GPS))
    final = pl.pallas_call(
        _combine_body,
        in_specs=[
            pl.BlockSpec(memory_space=pltpu.SMEM),
            pl.BlockSpec((_B_SC, _L), lambda: (0, 0)),
        ],
        out_specs=pl.BlockSpec(memory_space=pltpu.SMEM),
        out_shape=jax.ShapeDtypeStruct((1, 1), jnp.float32),
    )(out, scpart)
    return final[0, 0]
